# baseline (speedup 1.0000x reference)
import functools

import jax
import jax.numpy as jnp
from jax import lax
from jax.experimental import pallas as pl
from jax.experimental.pallas import tpu as pltpu

NUM_EMB = 8192
DIM = 256
ROWS = 8192
BLOCK = 256
NBLK = ROWS // BLOCK
PBLK = 1024 // BLOCK


def _vq_kernel(x_ref, w_ref, enc_ref, qst_ref, loss_ref, perp_ref,
               sw_ref, counts_ref, wbf_ref, loss_acc_ref):
    i = pl.program_id(0)

    @pl.when(i == 0)
    def _init():
        w = w_ref[...]
        sw_ref[...] = jnp.sum(w * w, axis=1, keepdims=True)
        counts_ref[...] = jnp.zeros((NUM_EMB, 1), jnp.float32)
        wbf_ref[...] = w.astype(jnp.bfloat16)
        loss_acc_ref[0, 0] = 0.0

    xc = x_ref[0]
    sx = jnp.sum(xc * xc, axis=0, keepdims=True)
    mm = lax.dot_general(w_ref[...], xc, (((1,), (0,)), ((), ())),
                         preferred_element_type=jnp.float32)
    d = (sx + sw_ref[...]) - 2.0 * mm
    dmin = jnp.min(d, axis=0, keepdims=True)
    iota0 = lax.broadcasted_iota(jnp.int32, (NUM_EMB, BLOCK), 0)
    idx = jnp.min(jnp.where(d == dmin, iota0, NUM_EMB), axis=0,
                  keepdims=True)
    enc_rt = (iota0 == idx).astype(jnp.float32).astype(jnp.bfloat16)

    q = lax.dot_general(wbf_ref[...], enc_rt, (((0,), (0,)), ((), ())),
                        preferred_element_type=jnp.float32)
    qst_ref[0] = xc - (q - xc)

    idx_col = jnp.transpose(idx, (1, 0))
    iota1 = lax.broadcasted_iota(jnp.int32, (BLOCK, NUM_EMB), 1)
    enc_ref[...] = (iota1 == idx_col).astype(jnp.float32)

    ones_bf = jnp.ones((BLOCK, 1), jnp.bfloat16)
    counts_ref[...] += lax.dot_general(
        enc_rt, ones_bf, (((1,), (0,)), ((), ())),
        preferred_element_type=jnp.float32)
    loss_acc_ref[0, 0] += jnp.sum(dmin)

    @pl.when(i == NBLK - 1)
    def _finalize():
        loss_ref[0, 0] = 1.25 * loss_acc_ref[0, 0] / (ROWS * DIM)
        p = jnp.transpose(counts_ref[...], (1, 0)) * (1.0 / ROWS)
        perp_ref[0, 0] = jnp.exp(-jnp.sum(p * jnp.log(p + 1e-10)))


@functools.partial(jax.jit)
def kernel(inputs, weight):
    x_cp = inputs.reshape(8, DIM, 1024)

    enc, qst, loss, perp = pl.pallas_call(
        _vq_kernel,
        grid=(NBLK,),
        in_specs=[
            pl.BlockSpec((1, DIM, BLOCK), lambda i: (i // PBLK, 0, i % PBLK)),
            pl.BlockSpec((NUM_EMB, DIM), lambda i: (0, 0)),
        ],
        out_specs=[
            pl.BlockSpec((BLOCK, NUM_EMB), lambda i: (i, 0)),
            pl.BlockSpec((1, DIM, BLOCK), lambda i: (i // PBLK, 0, i % PBLK)),
            pl.BlockSpec(memory_space=pltpu.SMEM),
            pl.BlockSpec(memory_space=pltpu.SMEM),
        ],
        out_shape=[
            jax.ShapeDtypeStruct((ROWS, NUM_EMB), jnp.float32),
            jax.ShapeDtypeStruct((8, DIM, 1024), jnp.float32),
            jax.ShapeDtypeStruct((1, 1), jnp.float32),
            jax.ShapeDtypeStruct((1, 1), jnp.float32),
        ],
        scratch_shapes=[
            pltpu.VMEM((NUM_EMB, 1), jnp.float32),
            pltpu.VMEM((NUM_EMB, 1), jnp.float32),
            pltpu.VMEM((NUM_EMB, DIM), jnp.bfloat16),
            pltpu.SMEM((1, 1), jnp.float32),
        ],
    )(x_cp, weight)

    quantized_st = qst.reshape(8, DIM, 32, 32)
    encodings = enc.reshape(ROWS, 1, NUM_EMB)
    return (loss[0, 0], quantized_st, perp[0, 0], encodings)


# speedup vs baseline: 1.5349x; 1.5349x over previous
import functools

import jax
import jax.numpy as jnp
from jax import lax
from jax.experimental import pallas as pl
from jax.experimental.pallas import tpu as pltpu

NUM_EMB = 8192
DIM = 256
ROWS = 8192
BLOCK = 256
NBLK = ROWS // BLOCK
PBLK = 1024 // BLOCK


def _vq_kernel(x_ref, w_ref, enc_ref, qst_ref, loss_ref, perp_ref,
               sw_ref, counts_ref, wbf_ref, loss_acc_ref):
    i = pl.program_id(0)

    @pl.when(i == 0)
    def _init():
        w = w_ref[...]
        sw_ref[...] = jnp.sum(w * w, axis=1, keepdims=True)
        counts_ref[...] = jnp.zeros((NUM_EMB, 1), jnp.float32)
        wbf_ref[...] = w.astype(jnp.bfloat16)
        loss_acc_ref[0, 0] = 0.0

    xc = x_ref[0]
    sx = jnp.sum(xc * xc, axis=0, keepdims=True)
    mm = lax.dot_general(w_ref[...], xc, (((1,), (0,)), ((), ())),
                         preferred_element_type=jnp.float32)
    d = (sx + sw_ref[...]) - 2.0 * mm
    dmin = jnp.min(d, axis=0, keepdims=True)
    iota0 = lax.broadcasted_iota(jnp.int32, (NUM_EMB, BLOCK), 0)
    idx = jnp.min(jnp.where(d == dmin, iota0, NUM_EMB), axis=0,
                  keepdims=True)
    enc_rt = (iota0 == idx).astype(jnp.float32).astype(jnp.bfloat16)

    q = lax.dot_general(wbf_ref[...], enc_rt, (((0,), (0,)), ((), ())),
                        preferred_element_type=jnp.float32)
    qst_ref[0] = xc - (q - xc)

    idx_col = jnp.transpose(idx, (1, 0))
    iota1 = lax.broadcasted_iota(jnp.int32, (BLOCK, NUM_EMB), 1)
    enc_ref[:, 0, :] = (iota1 == idx_col).astype(jnp.float32)

    ones_bf = jnp.ones((BLOCK, 1), jnp.bfloat16)
    counts_ref[...] += lax.dot_general(
        enc_rt, ones_bf, (((1,), (0,)), ((), ())),
        preferred_element_type=jnp.float32)
    loss_acc_ref[0, 0] += jnp.sum(dmin)

    @pl.when(i == NBLK - 1)
    def _finalize():
        loss_ref[0, 0] = 1.25 * loss_acc_ref[0, 0] / (ROWS * DIM)
        p = jnp.transpose(counts_ref[...], (1, 0)) * (1.0 / ROWS)
        perp_ref[0, 0] = jnp.exp(-jnp.sum(p * jnp.log(p + 1e-10)))


@functools.partial(jax.jit)
def kernel(inputs, weight):
    x_cp = inputs.reshape(8, DIM, 1024)

    enc, qst, loss, perp = pl.pallas_call(
        _vq_kernel,
        grid=(NBLK,),
        in_specs=[
            pl.BlockSpec((1, DIM, BLOCK), lambda i: (i // PBLK, 0, i % PBLK)),
            pl.BlockSpec((NUM_EMB, DIM), lambda i: (0, 0)),
        ],
        out_specs=[
            pl.BlockSpec((BLOCK, 1, NUM_EMB), lambda i: (i, 0, 0)),
            pl.BlockSpec((1, DIM, BLOCK), lambda i: (i // PBLK, 0, i % PBLK)),
            pl.BlockSpec(memory_space=pltpu.SMEM),
            pl.BlockSpec(memory_space=pltpu.SMEM),
        ],
        out_shape=[
            jax.ShapeDtypeStruct((ROWS, 1, NUM_EMB), jnp.float32),
            jax.ShapeDtypeStruct((8, DIM, 1024), jnp.float32),
            jax.ShapeDtypeStruct((1, 1), jnp.float32),
            jax.ShapeDtypeStruct((1, 1), jnp.float32),
        ],
        scratch_shapes=[
            pltpu.VMEM((NUM_EMB, 1), jnp.float32),
            pltpu.VMEM((NUM_EMB, 1), jnp.float32),
            pltpu.VMEM((NUM_EMB, DIM), jnp.bfloat16),
            pltpu.SMEM((1, 1), jnp.float32),
        ],
    )(x_cp, weight)

    quantized_st = qst.reshape(8, DIM, 32, 32)
    return (loss[0, 0], quantized_st, perp[0, 0], enc)
